# baseline (speedup 1.0000x reference)
import functools

import jax
import jax.numpy as jnp
from jax import lax
from jax.experimental import pallas as pl
from jax.experimental.pallas import tpu as pltpu
from jax.experimental.pallas import tpu_sc as plsc

_BT = 256
_A = 100000
_R = 8

_KS0 = 0
_KS1 = 42
_KS2 = 0 ^ 42 ^ 0x1BD11BDA
_KS = (_KS0, _KS1, _KS2)
_ROTS = ((13, 15, 26, 6), (17, 29, 16, 24))
_TINY = float(jnp.finfo(jnp.float32).tiny)


def _rotl(x, r):
    return lax.shift_left(x, r) | lax.shift_right_logical(x, 32 - r)


def _threefry_bits(i):
    x0 = jnp.full_like(i, _KS0)
    x1 = i + _KS1
    for r in range(5):
        for rot in _ROTS[r % 2]:
            x0 = x0 + x1
            x1 = _rotl(x1, rot)
            x1 = x1 ^ x0
        x0 = x0 + _KS[(r + 1) % 3]
        x1 = x1 + (_KS[(r + 2) % 3] + r + 1)
    return x0 ^ x1


def _sample_body(x_ref, lp_ref, idx_ref, alp_ref):
    b = pl.program_id(0)
    x = x_ref[...]

    row = lax.broadcasted_iota(jnp.int32, (_R, _A), 0) + b * _R
    col = lax.broadcasted_iota(jnp.int32, (_R, _A), 1)
    bits = _threefry_bits(row * _A + col)

    fbits = lax.shift_right_logical(bits, 9) | 0x3F800000
    f = lax.bitcast_convert_type(fbits, jnp.float32) - 1.0
    u = jnp.maximum(_TINY, f * (1.0 - _TINY) + _TINY)
    g = -jnp.log(-jnp.log(u))

    m = jnp.max(x, axis=1, keepdims=True)
    s = jnp.sum(jnp.exp(x - m), axis=1, keepdims=True)
    lp = x - (m + jnp.log(s))
    lp_ref[...] = lp

    idx = jnp.argmax(x + g, axis=1).astype(jnp.int32)
    idx_ref[...] = idx[:, None]
    sel = jnp.where(col == idx[:, None], lp, -jnp.inf)
    alp_ref[...] = jnp.max(sel, axis=1, keepdims=True)


def _sample(logits, interpret=False):
    return pl.pallas_call(
        _sample_body,
        grid=(_BT // _R,),
        in_specs=[pl.BlockSpec((_R, _A), lambda b: (b, 0))],
        out_specs=[
            pl.BlockSpec((_R, _A), lambda b: (b, 0)),
            pl.BlockSpec((_R, 1), lambda b: (b, 0)),
            pl.BlockSpec((_R, 1), lambda b: (b, 0)),
        ],
        out_shape=[
            jax.ShapeDtypeStruct((_BT, _A), jnp.float32),
            jax.ShapeDtypeStruct((_BT, 1), jnp.int32),
            jax.ShapeDtypeStruct((_BT, 1), jnp.float32),
        ],
        compiler_params=pltpu.CompilerParams(
            dimension_semantics=("parallel",),
        ),
        interpret=interpret,
    )(logits)


def kernel(logits, value, action_index_tensor):
    lp, idx, alp = _sample(logits)
    idx = idx.reshape(-1)
    actions = jnp.take(action_index_tensor, idx, axis=0)
    return actions, alp.reshape(-1), value.reshape(-1), lp
